# Initial kernel scaffold; baseline (speedup 1.0000x reference)
import jax
import jax.numpy as jnp
from jax.experimental import pallas as pl


def kernel(x):
    raise NotImplementedError("write your pallas kernel here")


# speedup vs baseline: 2.3249x; 2.3249x over previous
import functools

import jax
import jax.numpy as jnp
from jax import lax
from jax.experimental import pallas as pl
from jax.experimental.pallas import tpu as pltpu
from jax.experimental.pallas import tpu_sc as plsc

ROWS = 128
N = 32768
L = 16
NV = N // L
NBKT = 256
HSIZE = NBKT * L
MININT = -2**31


def _srl(x, amount):
    return lax.shift_right_logical(x, jnp.full((L,), amount, jnp.int32))


def _fwd_transform(ui):
    sign = _srl(ui, 31)
    return ui ^ ((0 - sign) | MININT)


def _inv_transform(k):
    top = _srl(k, 31)
    return k ^ ((top - 1) | MININT)


def _radix_pass(src_ref, dst_ref, h_ref, lane, ones, shift, first, last):
    zeros = jnp.zeros((L,), jnp.int32)

    @pl.loop(0, HSIZE, step=L)
    def _zero(i):
        h_ref[pl.ds(i, L)] = zeros

    def _key(i):
        k = src_ref[pl.ds(i, L)]
        if first:
            k = _fwd_transform(plsc.bitcast(k, jnp.int32))
        return k

    @pl.loop(0, N, step=L)
    def _hist(i):
        d = _srl(_key(i), shift) & 255
        plsc.addupdate_scatter(h_ref, [d * L + lane], ones)

    @pl.loop(0, HSIZE, step=L, init_carry=jnp.int32(0))
    def _scan(i, c):
        h = h_ref[pl.ds(i, L)]
        inc = plsc.cumsum(h)
        h_ref[pl.ds(i, L)] = inc - h + c
        return c + jnp.sum(h)

    @pl.loop(0, N, step=L)
    def _permute(i):
        k = _key(i)
        d = _srl(k, shift) & 255
        hidx = d * L + lane
        q = plsc.load_gather(h_ref, [hidx])
        plsc.store_scatter(h_ref, [hidx], q + 1)
        if last:
            val = plsc.bitcast(_inv_transform(k), jnp.float32)
            plsc.store_scatter(dst_ref, [q], val)
        else:
            phys = (q & (NV - 1)) * L + _srl(q, 11)
            plsc.store_scatter(dst_ref, [phys], k)


def _sort_body(x_hbm, o_hbm, a_ref, b_ref, c_ref, h_ref):
    cid = lax.axis_index("c")
    sid = lax.axis_index("s")
    wid = sid * 2 + cid
    lane = lax.iota(jnp.int32, L)
    ones = jnp.ones((L,), jnp.int32)
    rows_per_w = ROWS // 32

    for r in range(rows_per_w):
        row = wid * rows_per_w + r
        pltpu.sync_copy(x_hbm.at[row], a_ref)
        _radix_pass(a_ref, b_ref, h_ref, lane, ones, 0, first=True, last=False)
        _radix_pass(b_ref, c_ref, h_ref, lane, ones, 8, first=False, last=False)
        _radix_pass(c_ref, b_ref, h_ref, lane, ones, 16, first=False, last=False)
        _radix_pass(b_ref, a_ref, h_ref, lane, ones, 24, first=False, last=True)
        pltpu.sync_copy(a_ref, o_hbm.at[row])


def kernel(x):
    mesh = plsc.VectorSubcoreMesh(core_axis_name="c", subcore_axis_name="s")
    f = pl.kernel(
        _sort_body,
        out_type=jax.ShapeDtypeStruct((ROWS, N), jnp.float32),
        mesh=mesh,
        scratch_types=[
            pltpu.VMEM((N,), jnp.float32),
            pltpu.VMEM((N,), jnp.int32),
            pltpu.VMEM((N,), jnp.int32),
            pltpu.VMEM((HSIZE,), jnp.int32),
        ],
        compiler_params=pltpu.CompilerParams(needs_layout_passes=False),
    )
    return f(x)


# speedup vs baseline: 2.6139x; 1.1243x over previous
import functools

import jax
import jax.numpy as jnp
from jax import lax
from jax.experimental import pallas as pl
from jax.experimental.pallas import tpu as pltpu
from jax.experimental.pallas import tpu_sc as plsc

ROWS = 128
N = 32768
L = 16
NV = N // L
NBKT = 256
HSIZE = NBKT * L
MININT = -2**31


def _srl(x, amount):
    return lax.shift_right_logical(x, jnp.full((L,), amount, jnp.int32))


def _fwd_transform(ui):
    sign = _srl(ui, 31)
    return ui ^ ((0 - sign) | MININT)


def _inv_transform(k):
    top = _srl(k, 31)
    return k ^ ((top - 1) | MININT)


def _radix_pass(src_ref, dst_ref, h_ref, lane, ones, shift, first, last):
    zeros = jnp.zeros((L,), jnp.int32)

    @pl.loop(0, HSIZE, step=L, unroll=8)
    def _zero(i):
        h_ref[pl.ds(i, L)] = zeros

    def _key(i):
        k = src_ref[pl.ds(i, L)]
        if first:
            k = _fwd_transform(plsc.bitcast(k, jnp.int32))
        return k

    @pl.loop(0, N, step=L, unroll=8)
    def _hist(i):
        d = _srl(_key(i), shift) & 255
        plsc.addupdate_scatter(h_ref, [d * L + lane], ones)

    @pl.loop(0, HSIZE, step=L, init_carry=jnp.int32(0), unroll=8)
    def _scan(i, c):
        h = h_ref[pl.ds(i, L)]
        inc = plsc.cumsum(h)
        h_ref[pl.ds(i, L)] = inc - h + c
        return c + jnp.sum(h)

    @pl.loop(0, N, step=L, unroll=4)
    def _permute(i):
        k = _key(i)
        d = _srl(k, shift) & 255
        hidx = d * L + lane
        q = plsc.load_gather(h_ref, [hidx])
        plsc.store_scatter(h_ref, [hidx], q + 1)
        if last:
            val = plsc.bitcast(_inv_transform(k), jnp.float32)
            plsc.store_scatter(dst_ref, [q], val)
        else:
            phys = (q & (NV - 1)) * L + _srl(q, 11)
            plsc.store_scatter(dst_ref, [phys], k)


def _sort_body(x_hbm, o_hbm, a_ref, b_ref, c_ref, h_ref):
    cid = lax.axis_index("c")
    sid = lax.axis_index("s")
    wid = sid * 2 + cid
    lane = lax.iota(jnp.int32, L)
    ones = jnp.ones((L,), jnp.int32)
    rows_per_w = ROWS // 32

    for r in range(rows_per_w):
        row = wid * rows_per_w + r
        pltpu.sync_copy(x_hbm.at[row], a_ref)
        _radix_pass(a_ref, b_ref, h_ref, lane, ones, 0, first=True, last=False)
        _radix_pass(b_ref, c_ref, h_ref, lane, ones, 8, first=False, last=False)
        _radix_pass(c_ref, b_ref, h_ref, lane, ones, 16, first=False, last=False)
        _radix_pass(b_ref, a_ref, h_ref, lane, ones, 24, first=False, last=True)
        pltpu.sync_copy(a_ref, o_hbm.at[row])


def kernel(x):
    mesh = plsc.VectorSubcoreMesh(core_axis_name="c", subcore_axis_name="s")
    f = pl.kernel(
        _sort_body,
        out_type=jax.ShapeDtypeStruct((ROWS, N), jnp.float32),
        mesh=mesh,
        scratch_types=[
            pltpu.VMEM((N,), jnp.float32),
            pltpu.VMEM((N,), jnp.int32),
            pltpu.VMEM((N,), jnp.int32),
            pltpu.VMEM((HSIZE,), jnp.int32),
        ],
        compiler_params=pltpu.CompilerParams(needs_layout_passes=False),
    )
    return f(x)
